# baseline (speedup 1.0000x reference)
import functools

import jax
import jax.numpy as jnp
from jax import lax
from jax.experimental import pallas as pl
from jax.experimental.pallas import tpu as pltpu
from jax.experimental.pallas import tpu_sc as plsc

_N_TOKEN = 100000
_D_EMBED = 1024
_D_PROJ = 2048
_VEC_LEN = 128

_NC = 2
_NS = 16
_NW = _NC * _NS

_CHUNKS = (4096, 4096)
_BLOCK_M = 1024

_SL = 8
_NSLOT = 12


def _sc_gather(table, idx_full, chunk_base, chunk_rows):
  rows_per_w = chunk_rows // _NW
  n_sl = rows_per_w // _SL
  n_slot = min(_NSLOT, n_sl)
  mesh = plsc.VectorSubcoreMesh(
      core_axis_name="c", subcore_axis_name="s",
      num_cores=_NC, num_subcores=_NS)

  @functools.partial(
      pl.kernel,
      out_type=jax.ShapeDtypeStruct((chunk_rows, _D_EMBED), jnp.float32),
      mesh=mesh,
      scratch_types=[
          pltpu.VMEM((rows_per_w,), jnp.int32),
          pltpu.VMEM((n_slot * _SL, _D_EMBED), jnp.float32),
      ] + [pltpu.SemaphoreType.DMA] * (2 * n_slot),
  )
  def gather_kernel(table_hbm, idx_hbm, out_hbm, idx_v, rows_v, *sems):
    gsems = sems[:n_slot]
    wsems = sems[n_slot:]
    wid = lax.axis_index("s") * _NC + lax.axis_index("c")
    base = wid * rows_per_w
    pltpu.sync_copy(idx_hbm.at[pl.ds(chunk_base + base, rows_per_w)], idx_v)

    gcopies = [None] * n_slot
    wcopies = [None] * n_slot
    for i in range(n_slot):
      gcopies[i] = pltpu.async_copy(
          table_hbm.at[idx_v.at[pl.ds(i * _SL, _SL)]],
          rows_v.at[pl.ds(i * _SL, _SL)], gsems[i])
    for i in range(n_sl):
      j = i % n_slot
      gcopies[j].wait()
      wcopies[j] = pltpu.async_copy(
          rows_v.at[pl.ds(j * _SL, _SL)],
          out_hbm.at[pl.ds(base + i * _SL, _SL)], wsems[j])
      nxt = i + n_slot
      if nxt < n_sl:
        wcopies[j].wait()
        gcopies[j] = pltpu.async_copy(
            table_hbm.at[idx_v.at[pl.ds(nxt * _SL, _SL)]],
            rows_v.at[pl.ds(j * _SL, _SL)], gsems[j])
      else:
        wcopies[j].wait()

  return gather_kernel(table, idx_full)


def _proj_body(g_ref, s_ref, ws_ref, p_ref, o_ref):
  x = g_ref[...] + jnp.dot(
      s_ref[...], ws_ref[...], preferred_element_type=jnp.float32)
  acc = lax.dot_general(
      x.astype(jnp.bfloat16), p_ref[...], (((1,), (1,)), ((), ())),
      preferred_element_type=jnp.float32)
  o_ref[...] = acc * (_D_PROJ ** 0.5)


def _proj_kernel_first(g_ref, s_ref, ws_ref, p_ref, o_ref):
  _proj_body(g_ref, s_ref, ws_ref, p_ref, o_ref)


def _proj_kernel_next(o_in_ref, g_ref, s_ref, ws_ref, p_ref, o_ref):
  del o_in_ref
  _proj_body(g_ref, s_ref, ws_ref, p_ref, o_ref)


def _tc_project_chunk(out_buf, g, s_full, ws, p_bf, block_base, n_tok):
  blocks = g.shape[0] // _BLOCK_M
  specs = [
      pl.BlockSpec((_BLOCK_M, _D_EMBED), lambda i: (i, 0)),
      pl.BlockSpec((_BLOCK_M, _VEC_LEN), lambda i: (block_base + i, 0)),
      pl.BlockSpec((_VEC_LEN, _D_EMBED), lambda i: (0, 0)),
      pl.BlockSpec((_D_PROJ, _D_EMBED), lambda i: (0, 0)),
  ]
  out_spec = pl.BlockSpec((_BLOCK_M, _D_PROJ), lambda i: (block_base + i, 0))
  out_shape = jax.ShapeDtypeStruct((n_tok, _D_PROJ), jnp.float32)
  if out_buf is None:
    return pl.pallas_call(
        _proj_kernel_first,
        grid=(blocks,),
        in_specs=specs,
        out_specs=out_spec,
        out_shape=out_shape,
    )(g, s_full, ws, p_bf)
  return pl.pallas_call(
      _proj_kernel_next,
      grid=(blocks,),
      in_specs=[pl.BlockSpec(memory_space=pl.ANY)] + specs,
      out_specs=out_spec,
      out_shape=out_shape,
      input_output_aliases={0: 0},
  )(out_buf, g, s_full, ws, p_bf)


def kernel(inp, status_vec, emb_weight, status_weight, proj_W):
  b, l = inp.shape
  n_tok = b * l
  assert sum(_CHUNKS) == n_tok

  p_bf = proj_W.astype(jnp.bfloat16)
  idx_flat = inp.reshape(n_tok).astype(jnp.int32)
  s_flat = status_vec.reshape(n_tok, _VEC_LEN).astype(jnp.float32)
  ws_f32 = status_weight.astype(jnp.float32)

  bases = [sum(_CHUNKS[:k]) for k in range(len(_CHUNKS))]
  gathered = [
      _sc_gather(emb_weight, idx_flat, bases[k], _CHUNKS[k])
      for k in range(len(_CHUNKS))
  ]

  out = None
  for k in range(len(_CHUNKS)):
    out = _tc_project_chunk(out, gathered[k], s_flat, ws_f32, p_bf,
                            bases[k] // _BLOCK_M, n_tok)
  return out.reshape(b, l, _D_PROJ)


# speedup vs baseline: 1.0200x; 1.0200x over previous
import functools

import jax
import jax.numpy as jnp
from jax import lax
from jax.experimental import pallas as pl
from jax.experimental.pallas import tpu as pltpu
from jax.experimental.pallas import tpu_sc as plsc

_N_TOKEN = 100000
_D_EMBED = 1024
_D_PROJ = 2048
_VEC_LEN = 128

_NC = 2
_NS = 16
_NW = _NC * _NS

_CHUNKS = (3072, 5120)
_BLOCK_M = 1024

_SL = 8
_NSLOT = 12


def _sc_gather(table, idx_full, chunk_base, chunk_rows):
  rows_per_w = chunk_rows // _NW
  n_sl = rows_per_w // _SL
  n_slot = min(_NSLOT, n_sl)
  mesh = plsc.VectorSubcoreMesh(
      core_axis_name="c", subcore_axis_name="s",
      num_cores=_NC, num_subcores=_NS)

  @functools.partial(
      pl.kernel,
      out_type=jax.ShapeDtypeStruct((chunk_rows, _D_EMBED), jnp.float32),
      mesh=mesh,
      scratch_types=[
          pltpu.VMEM((rows_per_w,), jnp.int32),
          pltpu.VMEM((n_slot * _SL, _D_EMBED), jnp.float32),
      ] + [pltpu.SemaphoreType.DMA] * (2 * n_slot),
  )
  def gather_kernel(table_hbm, idx_hbm, out_hbm, idx_v, rows_v, *sems):
    gsems = sems[:n_slot]
    wsems = sems[n_slot:]
    wid = lax.axis_index("s") * _NC + lax.axis_index("c")
    base = wid * rows_per_w
    pltpu.sync_copy(idx_hbm.at[pl.ds(chunk_base + base, rows_per_w)], idx_v)

    gcopies = [None] * n_slot
    wcopies = [None] * n_slot
    for i in range(n_slot):
      gcopies[i] = pltpu.async_copy(
          table_hbm.at[idx_v.at[pl.ds(i * _SL, _SL)]],
          rows_v.at[pl.ds(i * _SL, _SL)], gsems[i])
    for i in range(n_sl):
      j = i % n_slot
      gcopies[j].wait()
      wcopies[j] = pltpu.async_copy(
          rows_v.at[pl.ds(j * _SL, _SL)],
          out_hbm.at[pl.ds(base + i * _SL, _SL)], wsems[j])
      nxt = i + n_slot
      if nxt < n_sl:
        wcopies[j].wait()
        gcopies[j] = pltpu.async_copy(
            table_hbm.at[idx_v.at[pl.ds(nxt * _SL, _SL)]],
            rows_v.at[pl.ds(j * _SL, _SL)], gsems[j])
      else:
        wcopies[j].wait()

  return gather_kernel(table, idx_full)


def _proj_body(g_ref, s_ref, ws_ref, p_ref, o_ref):
  x = g_ref[...] + jnp.dot(
      s_ref[...], ws_ref[...], preferred_element_type=jnp.float32)
  acc = lax.dot_general(
      x.astype(jnp.bfloat16), p_ref[...], (((1,), (1,)), ((), ())),
      preferred_element_type=jnp.float32)
  o_ref[...] = acc * (_D_PROJ ** 0.5)


def _proj_kernel_first(g_ref, s_ref, ws_ref, p_ref, o_ref):
  _proj_body(g_ref, s_ref, ws_ref, p_ref, o_ref)


def _proj_kernel_next(o_in_ref, g_ref, s_ref, ws_ref, p_ref, o_ref):
  del o_in_ref
  _proj_body(g_ref, s_ref, ws_ref, p_ref, o_ref)


def _tc_project_chunk(out_buf, g, s_full, ws, p_bf, block_base, n_tok):
  blocks = g.shape[0] // _BLOCK_M
  specs = [
      pl.BlockSpec((_BLOCK_M, _D_EMBED), lambda i: (i, 0)),
      pl.BlockSpec((_BLOCK_M, _VEC_LEN), lambda i: (block_base + i, 0)),
      pl.BlockSpec((_VEC_LEN, _D_EMBED), lambda i: (0, 0)),
      pl.BlockSpec((_D_PROJ, _D_EMBED), lambda i: (0, 0)),
  ]
  out_spec = pl.BlockSpec((_BLOCK_M, _D_PROJ), lambda i: (block_base + i, 0))
  out_shape = jax.ShapeDtypeStruct((n_tok, _D_PROJ), jnp.float32)
  if out_buf is None:
    return pl.pallas_call(
        _proj_kernel_first,
        grid=(blocks,),
        in_specs=specs,
        out_specs=out_spec,
        out_shape=out_shape,
    )(g, s_full, ws, p_bf)
  return pl.pallas_call(
      _proj_kernel_next,
      grid=(blocks,),
      in_specs=[pl.BlockSpec(memory_space=pl.ANY)] + specs,
      out_specs=out_spec,
      out_shape=out_shape,
      input_output_aliases={0: 0},
  )(out_buf, g, s_full, ws, p_bf)


def kernel(inp, status_vec, emb_weight, status_weight, proj_W):
  b, l = inp.shape
  n_tok = b * l
  assert sum(_CHUNKS) == n_tok

  p_bf = proj_W.astype(jnp.bfloat16)
  idx_flat = inp.reshape(n_tok).astype(jnp.int32)
  s_flat = status_vec.reshape(n_tok, _VEC_LEN).astype(jnp.float32)
  ws_f32 = status_weight.astype(jnp.float32)

  bases = [sum(_CHUNKS[:k]) for k in range(len(_CHUNKS))]
  gathered = [
      _sc_gather(emb_weight, idx_flat, bases[k], _CHUNKS[k])
      for k in range(len(_CHUNKS))
  ]

  out = None
  for k in range(len(_CHUNKS)):
    out = _tc_project_chunk(out, gathered[k], s_flat, ws_f32, p_bf,
                            bases[k] // _BLOCK_M, n_tok)
  return out.reshape(b, l, _D_PROJ)
